# baseline (speedup 1.0000x reference)
import functools

import jax
import jax.numpy as jnp
from jax import lax
from jax.experimental import pallas as pl
from jax.experimental.pallas import tpu as pltpu
from jax.experimental.pallas import tpu_sc as plsc

_N = 10000
_E = 320000
_D = 128
_H = 16
_K = 8

_NC = 2
_NS = 16
_NW = _NC * _NS
_EW = _E // _NW
_C = 2000
_NCH = _EW // _C
_SUB = 400
_NSUB = _C // _SUB
_G = _C // 16


def _tables_body(h_ref, w1_ref, b1_ref, w2_ref, b2_ref, w3_ref,
                 a_ref, b_ref, w3b_ref):
    h1 = lax.dot_general(h_ref[...], w1_ref[...],
                         (((1,), (1,)), ((), ())),
                         preferred_element_type=jnp.float32) + b1_ref[...][None, :]
    w2 = w2_ref[...]
    aval = lax.dot_general(h1, w2[:, :_H],
                           (((1,), (1,)), ((), ())),
                           preferred_element_type=jnp.float32) + b2_ref[...][None, :]
    lo = lax.convert_element_type(
        lax.bitcast_convert_type(aval[:, :4].astype(jnp.bfloat16), jnp.uint16),
        jnp.uint32)
    hi = lax.convert_element_type(
        lax.bitcast_convert_type(aval[:, 4:].astype(jnp.bfloat16), jnp.uint16),
        jnp.uint32)
    a_ref[...] = lax.bitcast_convert_type(lo | (hi << 16), jnp.int32)
    b_ref[...] = lax.dot_general(h1, w2[:, _H:],
                                 (((1,), (1,)), ((), ())),
                                 preferred_element_type=jnp.float32)
    w3b_ref[...] = lax.dot_general(w3_ref[...], jnp.ones((1, 16), jnp.float32),
                                   (((0,), (0,)), ((), ())),
                                   preferred_element_type=jnp.float32)


_tables_call = pl.pallas_call(
    _tables_body,
    out_shape=[
        jax.ShapeDtypeStruct((_N, _K // 2), jnp.int32),
        jax.ShapeDtypeStruct((_N, _K), jnp.float32),
        jax.ShapeDtypeStruct((_K, 16), jnp.float32),
    ],
)


def _edge_body(a_hbm, b_hbm, ei_hbm, w3_hbm, out_hbm,
               a_res, idx_s0, idx_s1, idx_d0, idx_d1,
               brows0, brows1, sco0, sco1, w3v,
               asem, isem, gsem, wsem):
    c = lax.axis_index("c")
    s = lax.axis_index("s")
    wid = s * _NC + c
    base = pl.multiple_of(wid * _EW, 256)

    pltpu.sync_copy(w3_hbm, w3v)
    iot = lax.iota(jnp.int32, 16)
    kd = [(iot + cc) & (_K - 1) for cc in range(_K)]
    w3d = [plsc.load_gather(w3v, [kd[cc], iot]) for cc in range(_K)]
    wd = [(iot + c4) & 3 for c4 in range(4)]
    hsel = [kd[cc] >= 4 for cc in range(_K)]

    astage = pltpu.async_copy(a_hbm, a_res, asem)

    sbufs = [idx_s0, idx_s1]
    dbufs = [idx_d0, idx_d1]
    bbufs = [brows0, brows1]
    obufs = [sco0, sco1]

    def issue_idx(ch):
        b = ch % 2
        return (pltpu.async_copy(ei_hbm.at[0, pl.ds(base + ch * _C, _C)],
                                 sbufs[b], isem),
                pltpu.async_copy(ei_hbm.at[1, pl.ds(base + ch * _C, _C)],
                                 dbufs[b], isem))

    def issue_gathers(ch):
        b = ch % 2
        cps = []
        for j in range(_NSUB):
            cps.append(pltpu.async_copy(
                b_hbm.at[dbufs[b].at[pl.ds(j * _SUB, _SUB)]],
                bbufs[b].at[pl.ds(j * _SUB, _SUB)], gsem))
        return cps

    def compute(ch):
        b = ch % 2
        sb, br, sc = sbufs[b], bbufs[b], obufs[b]

        def group(g, gcarry):
            sv = sb[pl.ds(pl.multiple_of(g * 16, 16), 16)]
            rows = g * 16 + iot
            xs = [plsc.load_gather(a_res, [sv, wd[c4]]) for c4 in range(4)]
            alo = [plsc.bitcast(x << 16, jnp.float32) for x in xs]
            ahi = [plsc.bitcast(x & jnp.int32(-65536), jnp.float32) for x in xs]
            terms = []
            for cc in range(_K):
                av = jnp.where(hsel[cc], ahi[cc % 4], alo[cc % 4])
                bv = plsc.load_gather(br, [rows, kd[cc]])
                terms.append(jnp.maximum(av + bv, 0.0) * w3d[cc])
            t01 = terms[0] + terms[1]
            t23 = terms[2] + terms[3]
            t45 = terms[4] + terms[5]
            t67 = terms[6] + terms[7]
            sc[pl.ds(pl.multiple_of(g * 16, 16), 16)] = (t01 + t23) + (t45 + t67)
            return gcarry

        lax.fori_loop(0, _G, group, 0)
        return pltpu.async_copy(sc, out_hbm.at[pl.ds(base + ch * _C, _C)],
                                wsem)

    idxcp = {0: issue_idx(0)}
    for cp in idxcp[0]:
        cp.wait()
    gath = {0: issue_gathers(0)}
    idxcp[1] = issue_idx(1)
    astage.wait()

    writes = []
    for ch in range(_NCH):
        if ch + 1 < _NCH:
            for cp in idxcp[ch + 1]:
                cp.wait()
            gath[ch + 1] = issue_gathers(ch + 1)
        for cp in gath[ch]:
            cp.wait()
        if ch >= 2:
            writes[ch - 2].wait()
        writes.append(compute(ch))
        if ch + 2 < _NCH:
            idxcp[ch + 2] = issue_idx(ch + 2)
    writes[-2].wait()
    writes[-1].wait()


_edge_call = functools.partial(
    pl.kernel,
    out_type=jax.ShapeDtypeStruct((_E,), jnp.float32),
    mesh=plsc.VectorSubcoreMesh(core_axis_name="c", subcore_axis_name="s",
                                num_cores=_NC, num_subcores=_NS),
    compiler_params=pltpu.CompilerParams(
        needs_layout_passes=False, use_tc_tiling_on_sc=False),
    scratch_types=[
        pltpu.VMEM((_N, _K // 2), jnp.int32),
        pltpu.VMEM((_C,), jnp.int32),
        pltpu.VMEM((_C,), jnp.int32),
        pltpu.VMEM((_C,), jnp.int32),
        pltpu.VMEM((_C,), jnp.int32),
        pltpu.VMEM((_C, _K), jnp.float32),
        pltpu.VMEM((_C, _K), jnp.float32),
        pltpu.VMEM((_C,), jnp.float32),
        pltpu.VMEM((_C,), jnp.float32),
        pltpu.VMEM((_K, 16), jnp.float32),
        pltpu.SemaphoreType.DMA,
        pltpu.SemaphoreType.DMA,
        pltpu.SemaphoreType.DMA,
        pltpu.SemaphoreType.DMA,
    ],
)(_edge_body)


def _norm_body(s_ref, o_ref):
    sv = s_ref[...]
    mn = jnp.min(sv)
    mx = jnp.max(sv)
    o_ref[...] = (sv - mn) / (mx - mn)


_norm_call = pl.pallas_call(
    _norm_body,
    out_shape=jax.ShapeDtypeStruct((_E,), jnp.float32),
)


def kernel(h, edge_index, W1, b1, W2, b2, W3, b3):
    a_tab, b_tab, w3b = _tables_call(h, W1, b1, W2, b2, W3)
    scores = _edge_call(a_tab, b_tab, edge_index, w3b)
    return _norm_call(scores).reshape(_E, 1)


# speedup vs baseline: 1.0035x; 1.0035x over previous
import functools

import jax
import jax.numpy as jnp
from jax import lax
from jax.experimental import pallas as pl
from jax.experimental.pallas import tpu as pltpu
from jax.experimental.pallas import tpu_sc as plsc

_N = 10000
_E = 320000
_D = 128
_H = 16
_K = 8

_NC = 2
_NS = 16
_NW = _NC * _NS
_EW = _E // _NW
_C = 2000
_NCH = _EW // _C
_SUB = 400
_NSUB = _C // _SUB
_G = _C // 16


def _tables_body(h_ref, w1_ref, b1_ref, w2_ref, b2_ref, w3_ref,
                 a_ref, b_ref, w3b_ref):
    h1 = lax.dot_general(h_ref[...], w1_ref[...],
                         (((1,), (1,)), ((), ())),
                         preferred_element_type=jnp.float32) + b1_ref[...][None, :]
    w2 = w2_ref[...]
    a_ref[...] = lax.dot_general(h1, w2[:, :_H],
                                 (((1,), (1,)), ((), ())),
                                 preferred_element_type=jnp.float32) + b2_ref[...][None, :]
    b_ref[...] = lax.dot_general(h1, w2[:, _H:],
                                 (((1,), (1,)), ((), ())),
                                 preferred_element_type=jnp.float32)
    w3b_ref[...] = lax.dot_general(w3_ref[...], jnp.ones((1, 16), jnp.float32),
                                   (((0,), (0,)), ((), ())),
                                   preferred_element_type=jnp.float32)


_tables_call = pl.pallas_call(
    _tables_body,
    out_shape=[
        jax.ShapeDtypeStruct((_N, _K), jnp.float32),
        jax.ShapeDtypeStruct((_N, _K), jnp.float32),
        jax.ShapeDtypeStruct((_K, 16), jnp.float32),
    ],
)


def _edge_body(a_hbm, b_hbm, ei_hbm, w3_hbm, out_hbm,
               a_res, idx_s0, idx_s1, idx_d0, idx_d1,
               brows0, brows1, sco0, sco1, w3v,
               asem, isem, gsem, wsem):
    c = lax.axis_index("c")
    s = lax.axis_index("s")
    wid = s * _NC + c
    base = pl.multiple_of(wid * _EW, 256)

    pltpu.sync_copy(w3_hbm, w3v)
    iot = lax.iota(jnp.int32, 16)
    kd = [(iot + cc) & (_K - 1) for cc in range(_K)]
    w3d = [plsc.load_gather(w3v, [kd[cc], iot]) for cc in range(_K)]

    astage = pltpu.async_copy(a_hbm, a_res, asem)

    sbufs = [idx_s0, idx_s1]
    dbufs = [idx_d0, idx_d1]
    bbufs = [brows0, brows1]
    obufs = [sco0, sco1]

    def issue_idx(ch):
        b = ch % 2
        return (pltpu.async_copy(ei_hbm.at[0, pl.ds(base + ch * _C, _C)],
                                 sbufs[b], isem),
                pltpu.async_copy(ei_hbm.at[1, pl.ds(base + ch * _C, _C)],
                                 dbufs[b], isem))

    def issue_gathers(ch):
        b = ch % 2
        cps = []
        for j in range(_NSUB):
            cps.append(pltpu.async_copy(
                b_hbm.at[dbufs[b].at[pl.ds(j * _SUB, _SUB)]],
                bbufs[b].at[pl.ds(j * _SUB, _SUB)], gsem))
        return cps

    def compute(ch):
        b = ch % 2
        sb, br, sc = sbufs[b], bbufs[b], obufs[b]

        def group(g, gcarry):
            sv = sb[pl.ds(pl.multiple_of(g * 16, 16), 16)]
            rows = g * 16 + iot
            terms = []
            for cc in range(_K):
                av = plsc.load_gather(a_res, [sv, kd[cc]])
                bv = plsc.load_gather(br, [rows, kd[cc]])
                terms.append(jnp.maximum(av + bv, 0.0) * w3d[cc])
            t01 = terms[0] + terms[1]
            t23 = terms[2] + terms[3]
            t45 = terms[4] + terms[5]
            t67 = terms[6] + terms[7]
            sc[pl.ds(pl.multiple_of(g * 16, 16), 16)] = (t01 + t23) + (t45 + t67)
            return gcarry

        lax.fori_loop(0, _G, group, 0)
        return pltpu.async_copy(sc, out_hbm.at[pl.ds(base + ch * _C, _C)],
                                wsem)

    idxcp = {0: issue_idx(0)}
    for cp in idxcp[0]:
        cp.wait()
    gath = {0: issue_gathers(0)}
    idxcp[1] = issue_idx(1)
    astage.wait()

    writes = []
    for ch in range(_NCH):
        if ch + 1 < _NCH:
            for cp in idxcp[ch + 1]:
                cp.wait()
            gath[ch + 1] = issue_gathers(ch + 1)
        for cp in gath[ch]:
            cp.wait()
        if ch >= 2:
            writes[ch - 2].wait()
        writes.append(compute(ch))
        if ch + 2 < _NCH:
            idxcp[ch + 2] = issue_idx(ch + 2)
    writes[-2].wait()
    writes[-1].wait()


_edge_call = functools.partial(
    pl.kernel,
    out_type=jax.ShapeDtypeStruct((_E,), jnp.float32),
    mesh=plsc.VectorSubcoreMesh(core_axis_name="c", subcore_axis_name="s",
                                num_cores=_NC, num_subcores=_NS),
    compiler_params=pltpu.CompilerParams(
        needs_layout_passes=False, use_tc_tiling_on_sc=False),
    scratch_types=[
        pltpu.VMEM((_N, _K), jnp.float32),
        pltpu.VMEM((_C,), jnp.int32),
        pltpu.VMEM((_C,), jnp.int32),
        pltpu.VMEM((_C,), jnp.int32),
        pltpu.VMEM((_C,), jnp.int32),
        pltpu.VMEM((_C, _K), jnp.float32),
        pltpu.VMEM((_C, _K), jnp.float32),
        pltpu.VMEM((_C,), jnp.float32),
        pltpu.VMEM((_C,), jnp.float32),
        pltpu.VMEM((_K, 16), jnp.float32),
        pltpu.SemaphoreType.DMA,
        pltpu.SemaphoreType.DMA,
        pltpu.SemaphoreType.DMA,
        pltpu.SemaphoreType.DMA,
    ],
)(_edge_body)


def _norm_body(s_ref, o_ref):
    sv = s_ref[...]
    mn = jnp.min(sv)
    mx = jnp.max(sv)
    o_ref[...] = (sv - mn) / (mx - mn)


_norm_call = pl.pallas_call(
    _norm_body,
    out_shape=jax.ShapeDtypeStruct((_E,), jnp.float32),
)


def kernel(h, edge_index, W1, b1, W2, b2, W3, b3):
    a_tab, b_tab, w3b = _tables_call(h, W1, b1, W2, b2, W3)
    scores = _edge_call(a_tab, b_tab, edge_index, w3b)
    return _norm_call(scores).reshape(_E, 1)
